# baseline (speedup 1.0000x reference)
import functools

import jax
import jax.numpy as jnp
from jax import lax
from jax.experimental import pallas as pl
from jax.experimental.pallas import tpu as pltpu
from jax.experimental.pallas import tpu_sc as plsc

D_MODEL = 64
BATCH = 4096
HIST = 200
TAB_ROWS = 3001
TAB_PAD = 3072

_info = plsc.get_sparse_core_info()
NC = _info.num_cores
NS = _info.num_subcores
NW = NC * NS
N_CGRP = 8
N_NGRP = NW // N_CGRP
COLS_W = D_MODEL // N_CGRP
N_PER_W = HIST // N_NGRP
BV = BATCH // 128


def _make_gather():
  mesh = plsc.VectorSubcoreMesh(core_axis_name="c", subcore_axis_name="s")

  @functools.partial(
      pl.kernel,
      mesh=mesh,
      out_type=jax.ShapeDtypeStruct((HIST, D_MODEL, 1, BATCH), jnp.float32),
      compiler_params=pltpu.CompilerParams(
          needs_layout_passes=False, use_tc_tiling_on_sc=False),
      scratch_types=[
          pltpu.VMEM((COLS_W * TAB_PAD,), jnp.float32),
          pltpu.VMEM((BATCH,), jnp.int32),
          pltpu.VMEM((COLS_W, 1, BATCH), jnp.float32),
          pltpu.SemaphoreType.DMA,
      ],
  )
  def gather_kernel(idx_hbm, pet_hbm, out_hbm, colbuf, idxbuf, outbuf, sem):
    wid = lax.axis_index("s") * NC + lax.axis_index("c")
    c0 = (wid % N_CGRP) * COLS_W
    n0 = (wid // N_CGRP) * N_PER_W

    for c in range(COLS_W):
      pltpu.sync_copy(pet_hbm.at[c0 + c],
                      colbuf.at[pl.ds(c * TAB_PAD, TAB_PAD)])

    def step(n_i, carry):
      n = n0 + n_i
      pltpu.sync_copy(idx_hbm.at[pl.ds(n * BATCH, BATCH)], idxbuf)

      def blk(bb, inner):
        for j in range(8):
          iv = idxbuf[pl.ds(bb * 128 + 16 * j, 16)]
          for c in range(COLS_W):
            g = plsc.load_gather(colbuf, [iv + jnp.int32(c * TAB_PAD)])
            outbuf[c, 0, pl.ds(bb * 128 + 16 * j, 16)] = g
        return inner

      lax.fori_loop(0, BV, blk, 0)
      pltpu.sync_copy(outbuf, out_hbm.at[n, pl.ds(c0, COLS_W)])
      return carry

    lax.fori_loop(0, N_PER_W, step, 0)

  return gather_kernel


_gather = _make_gather()


def kernel(doy, pe):
  idx_nb = doy.T.reshape(HIST * BATCH)
  pe_t = jnp.pad(pe.T, ((0, 0), (0, TAB_PAD - TAB_ROWS)))
  out = _gather(idx_nb, pe_t)
  return jnp.transpose(out, (3, 0, 1, 2))[:, :, :, :, None]


# speedup vs baseline: 1.2593x; 1.2593x over previous
import functools

import jax
import jax.numpy as jnp
from jax import lax
from jax.experimental import pallas as pl
from jax.experimental.pallas import tpu as pltpu
from jax.experimental.pallas import tpu_sc as plsc

D_MODEL = 64
BATCH = 4096
HIST = 200
TAB_ROWS = 3001
TAB_PAD = 3072

_info = plsc.get_sparse_core_info()
NC = _info.num_cores
NS = _info.num_subcores
NW = NC * NS
N_CGRP = 8
N_NGRP = NW // N_CGRP
COLS_W = D_MODEL // N_CGRP
N_PER_W = HIST // N_NGRP


def _make_gather():
  mesh = plsc.VectorSubcoreMesh(core_axis_name="c", subcore_axis_name="s")

  @functools.partial(
      pl.kernel,
      mesh=mesh,
      out_type=jax.ShapeDtypeStruct((HIST, D_MODEL, 1, BATCH), jnp.float32),
      compiler_params=pltpu.CompilerParams(
          needs_layout_passes=False, use_tc_tiling_on_sc=False),
      scratch_types=[
          pltpu.VMEM((COLS_W * TAB_PAD,), jnp.float32),
          pltpu.VMEM((BATCH,), jnp.int32),
          pltpu.VMEM((BATCH,), jnp.int32),
          pltpu.VMEM((COLS_W, 1, BATCH), jnp.float32),
          pltpu.VMEM((COLS_W, 1, BATCH), jnp.float32),
          pltpu.SemaphoreType.DMA,
          pltpu.SemaphoreType.DMA,
          pltpu.SemaphoreType.DMA,
          pltpu.SemaphoreType.DMA,
      ],
  )
  def gather_kernel(idx_hbm, pet_hbm, out_hbm, colbuf, idxb0, idxb1,
                    outb0, outb1, s_i0, s_i1, s_o0, s_o1):
    wid = lax.axis_index("s") * NC + lax.axis_index("c")
    c0 = (wid % N_CGRP) * COLS_W
    n0 = (wid // N_CGRP) * N_PER_W

    for c in range(COLS_W):
      pltpu.sync_copy(pet_hbm.at[c0 + c],
                      colbuf.at[pl.ds(c * TAB_PAD, TAB_PAD)])

    slots = ((idxb0, outb0, s_i0, s_o0), (idxb1, outb1, s_i1, s_o1))

    def idx_src(n):
      return idx_hbm.at[pl.ds(n * BATCH, BATCH)]

    def out_dst(n):
      return out_hbm.at[n, pl.ds(c0, COLS_W)]

    pltpu.async_copy(idx_src(n0), idxb0, s_i0)

    def pair(j, carry):
      for b in range(2):
        idxb, outb, s_i, s_o = slots[b]
        o_idxb, _, o_s_i, _ = slots[1 - b]
        n_i = 2 * j + b
        n = n0 + n_i

        pltpu.make_async_copy(idx_src(n), idxb, s_i).wait()

        if b == 0:
          pltpu.async_copy(idx_src(n + 1), o_idxb, o_s_i)
        else:
          @pl.when(j < N_PER_W // 2 - 1)
          def _():
            pltpu.async_copy(idx_src(n + 1), o_idxb, o_s_i)

        @pl.when(j >= 1)
        def _():
          pltpu.make_async_copy(outb, out_dst(n - 2), s_o).wait()

        def blk(bb, inner):
          for jj in range(8):
            iv = idxb[pl.ds(bb * 128 + 16 * jj, 16)]
            for c in range(COLS_W):
              g = plsc.load_gather(colbuf, [iv + jnp.int32(c * TAB_PAD)])
              outb[c, 0, pl.ds(bb * 128 + 16 * jj, 16)] = g
          return inner

        lax.fori_loop(0, BATCH // 128, blk, 0)
        pltpu.async_copy(outb, out_dst(n), s_o)
      return carry

    lax.fori_loop(0, N_PER_W // 2, pair, 0)

    pltpu.make_async_copy(outb0, out_dst(n0 + N_PER_W - 2), s_o0).wait()
    pltpu.make_async_copy(outb1, out_dst(n0 + N_PER_W - 1), s_o1).wait()

  return gather_kernel


_gather = _make_gather()


def kernel(doy, pe):
  idx_nb = doy.T.reshape(HIST * BATCH)
  pe_t = jnp.pad(pe.T, ((0, 0), (0, TAB_PAD - TAB_ROWS)))
  out = _gather(idx_nb, pe_t)
  return jnp.transpose(out, (3, 0, 1, 2))[:, :, :, :, None]


# speedup vs baseline: 2.6894x; 2.1357x over previous
import functools

import jax
import jax.numpy as jnp
from jax import lax
from jax.experimental import pallas as pl
from jax.experimental.pallas import tpu as pltpu
from jax.experimental.pallas import tpu_sc as plsc

D_MODEL = 64
BATCH = 4096
HIST = 200
TAB_ROWS = 3001
TAB_PAD = 3072

_info = plsc.get_sparse_core_info()
NC = _info.num_cores
NS = _info.num_subcores
NW = NC * NS
N_CGRP = 8
N_NGRP = NW // N_CGRP
COLS_W = D_MODEL // N_CGRP
N_PER_W = HIST // N_NGRP


def _make_gather():
  mesh = plsc.VectorSubcoreMesh(core_axis_name="c", subcore_axis_name="s")

  @functools.partial(
      pl.kernel,
      mesh=mesh,
      out_type=jax.ShapeDtypeStruct((HIST, D_MODEL, 1, BATCH), jnp.float32),
      compiler_params=pltpu.CompilerParams(
          needs_layout_passes=False, use_tc_tiling_on_sc=False),
      scratch_types=[
          pltpu.VMEM((COLS_W * TAB_PAD,), jnp.float32),
          pltpu.VMEM((BATCH,), jnp.int32),
          pltpu.VMEM((BATCH,), jnp.int32),
          pltpu.VMEM((COLS_W, 1, BATCH), jnp.float32),
          pltpu.VMEM((COLS_W, 1, BATCH), jnp.float32),
          pltpu.SemaphoreType.DMA,
          pltpu.SemaphoreType.DMA,
          pltpu.SemaphoreType.DMA,
          pltpu.SemaphoreType.DMA,
      ],
  )
  def gather_kernel(idx_hbm, pet_hbm, out_hbm, colbuf, idxb0, idxb1,
                    outb0, outb1, s_i0, s_i1, s_o0, s_o1):
    wid = lax.axis_index("s") * NC + lax.axis_index("c")
    c0 = (wid % N_CGRP) * COLS_W
    n0 = (wid // N_CGRP) * N_PER_W

    for c in range(COLS_W):
      pltpu.sync_copy(pet_hbm.at[c0 + c],
                      colbuf.at[pl.ds(c * TAB_PAD, TAB_PAD)])

    slots = ((idxb0, outb0, s_i0, s_o0), (idxb1, outb1, s_i1, s_o1))

    def idx_src(n):
      return idx_hbm.at[pl.ds(n * BATCH, BATCH)]

    def out_dst(n):
      return out_hbm.at[n, pl.ds(c0, COLS_W)]

    pltpu.async_copy(idx_src(n0), idxb0, s_i0)

    def pair(j, carry):
      for b in range(2):
        idxb, outb, s_i, s_o = slots[b]
        o_idxb, _, o_s_i, _ = slots[1 - b]
        n_i = 2 * j + b
        n = n0 + n_i

        pltpu.make_async_copy(idx_src(n), idxb, s_i).wait()

        if b == 0:
          pltpu.async_copy(idx_src(n + 1), o_idxb, o_s_i)
        else:
          @pl.when(j < N_PER_W // 2 - 1)
          def _():
            pltpu.async_copy(idx_src(n + 1), o_idxb, o_s_i)

        @pl.when(j >= 1)
        def _():
          pltpu.make_async_copy(outb, out_dst(n - 2), s_o).wait()

        def blk(bb, inner):
          for jj in range(8):
            iv = idxb[pl.ds(bb * 128 + 16 * jj, 16)]
            ivc = [iv + jnp.int32(c * TAB_PAD) for c in range(COLS_W)]
            gs = [plsc.load_gather(colbuf, [ivc[c]]) for c in range(COLS_W)]
            for c in range(COLS_W):
              outb[c, 0, pl.ds(bb * 128 + 16 * jj, 16)] = gs[c]
          return inner

        lax.fori_loop(0, BATCH // 128, blk, 0)
        pltpu.async_copy(outb, out_dst(n), s_o)
      return carry

    lax.fori_loop(0, N_PER_W // 2, pair, 0)

    pltpu.make_async_copy(outb0, out_dst(n0 + N_PER_W - 2), s_o0).wait()
    pltpu.make_async_copy(outb1, out_dst(n0 + N_PER_W - 1), s_o1).wait()

  return gather_kernel


_gather = _make_gather()


def kernel(doy, pe):
  idx_nb = doy.T.reshape(HIST * BATCH)
  pe_t = jnp.pad(pe.T, ((0, 0), (0, TAB_PAD - TAB_ROWS)))
  out = _gather(idx_nb, pe_t)
  return jnp.transpose(out, (3, 0, 1, 2))[:, :, :, :, None]


# speedup vs baseline: 2.9470x; 1.0957x over previous
import functools

import jax
import jax.numpy as jnp
from jax import lax
from jax.experimental import pallas as pl
from jax.experimental.pallas import tpu as pltpu
from jax.experimental.pallas import tpu_sc as plsc

D_MODEL = 64
BATCH = 4096
HIST = 200
TAB_ROWS = 3001
TAB_PAD = 3072

_info = plsc.get_sparse_core_info()
NC = _info.num_cores
NS = _info.num_subcores
NW = NC * NS
N_CGRP = 8
N_NGRP = NW // N_CGRP
COLS_W = D_MODEL // N_CGRP
N_PER_W = HIST // N_NGRP


def _make_gather():
  mesh = plsc.VectorSubcoreMesh(core_axis_name="c", subcore_axis_name="s")

  @functools.partial(
      pl.kernel,
      mesh=mesh,
      out_type=jax.ShapeDtypeStruct((HIST, D_MODEL, 1, BATCH), jnp.float32),
      compiler_params=pltpu.CompilerParams(
          needs_layout_passes=False, use_tc_tiling_on_sc=False),
      scratch_types=[
          pltpu.VMEM((COLS_W * TAB_PAD,), jnp.float32),
          pltpu.VMEM((BATCH,), jnp.int32),
          pltpu.VMEM((BATCH,), jnp.int32),
          pltpu.VMEM((COLS_W, 1, BATCH), jnp.float32),
          pltpu.VMEM((COLS_W, 1, BATCH), jnp.float32),
          pltpu.SemaphoreType.DMA,
          pltpu.SemaphoreType.DMA,
          pltpu.SemaphoreType.DMA,
          pltpu.SemaphoreType.DMA,
      ],
  )
  def gather_kernel(idx_hbm, pet_hbm, out_hbm, colbuf, idxb0, idxb1,
                    outb0, outb1, s_i0, s_i1, s_o0, s_o1):
    wid = lax.axis_index("s") * NC + lax.axis_index("c")
    c0 = (wid % N_CGRP) * COLS_W
    n0 = (wid // N_CGRP) * N_PER_W

    for c in range(COLS_W):
      pltpu.sync_copy(pet_hbm.at[c0 + c],
                      colbuf.at[pl.ds(c * TAB_PAD, TAB_PAD)])

    slots = ((idxb0, outb0, s_i0, s_o0), (idxb1, outb1, s_i1, s_o1))

    def idx_src(n):
      return idx_hbm.at[pl.ds(n * BATCH, BATCH)]

    def out_dst(n):
      return out_hbm.at[n, pl.ds(c0, COLS_W)]

    pltpu.async_copy(idx_src(n0), idxb0, s_i0)

    def pair(j, carry):
      for b in range(2):
        idxb, outb, s_i, s_o = slots[b]
        o_idxb, _, o_s_i, _ = slots[1 - b]
        n_i = 2 * j + b
        n = n0 + n_i

        pltpu.make_async_copy(idx_src(n), idxb, s_i).wait()

        if b == 0:
          pltpu.async_copy(idx_src(n + 1), o_idxb, o_s_i)
        else:
          @pl.when(j < N_PER_W // 2 - 1)
          def _():
            pltpu.async_copy(idx_src(n + 1), o_idxb, o_s_i)

        @pl.when(j >= 1)
        def _():
          pltpu.make_async_copy(outb, out_dst(n - 2), s_o).wait()

        @plsc.parallel_loop(0, BATCH, step=128, unroll=2)
        def _blk(base):
          for jj in range(8):
            iv = idxb[pl.ds(base + 16 * jj, 16)]
            ivc = [iv + jnp.int32(c * TAB_PAD) for c in range(COLS_W)]
            gs = [plsc.load_gather(colbuf, [ivc[c]]) for c in range(COLS_W)]
            for c in range(COLS_W):
              outb[c, 0, pl.ds(base + 16 * jj, 16)] = gs[c]
        pltpu.async_copy(outb, out_dst(n), s_o)
      return carry

    lax.fori_loop(0, N_PER_W // 2, pair, 0)

    pltpu.make_async_copy(outb0, out_dst(n0 + N_PER_W - 2), s_o0).wait()
    pltpu.make_async_copy(outb1, out_dst(n0 + N_PER_W - 1), s_o1).wait()

  return gather_kernel


_gather = _make_gather()


def kernel(doy, pe):
  idx_nb = doy.T.reshape(HIST * BATCH)
  pe_t = jnp.pad(pe.T, ((0, 0), (0, TAB_PAD - TAB_ROWS)))
  out = _gather(idx_nb, pe_t)
  return jnp.transpose(out, (3, 0, 1, 2))[:, :, :, :, None]


# speedup vs baseline: 3.5784x; 1.2143x over previous
import functools

import jax
import jax.numpy as jnp
from jax import lax
from jax.experimental import pallas as pl
from jax.experimental.pallas import tpu as pltpu
from jax.experimental.pallas import tpu_sc as plsc

D_MODEL = 64
BATCH = 4096
HIST = 200
TAB_ROWS = 3001
TAB_PAD = 3072

_info = plsc.get_sparse_core_info()
NC = _info.num_cores
NS = _info.num_subcores
NW = NC * NS
N_CGRP = 8
N_NGRP = NW // N_CGRP
COLS_W = D_MODEL // N_CGRP
N_PER_W = HIST // N_NGRP


def _make_gather():
  mesh = plsc.VectorSubcoreMesh(core_axis_name="c", subcore_axis_name="s")

  @functools.partial(
      pl.kernel,
      mesh=mesh,
      out_type=jax.ShapeDtypeStruct((HIST, D_MODEL, 1, BATCH), jnp.float32),
      compiler_params=pltpu.CompilerParams(
          needs_layout_passes=False, use_tc_tiling_on_sc=False),
      scratch_types=[
          pltpu.VMEM((COLS_W * TAB_PAD,), jnp.float32),
          pltpu.VMEM((BATCH,), jnp.int32),
          pltpu.VMEM((BATCH,), jnp.int32),
          pltpu.VMEM((COLS_W, 1, BATCH), jnp.float32),
          pltpu.VMEM((COLS_W, 1, BATCH), jnp.float32),
          pltpu.SemaphoreType.DMA,
          pltpu.SemaphoreType.DMA,
          pltpu.SemaphoreType.DMA,
          pltpu.SemaphoreType.DMA,
      ],
  )
  def gather_kernel(idx_hbm, pet_hbm, out_hbm, colbuf, idxb0, idxb1,
                    outb0, outb1, s_i0, s_i1, s_o0, s_o1):
    wid = lax.axis_index("s") * NC + lax.axis_index("c")
    c0 = (wid % N_CGRP) * COLS_W
    n0 = (wid // N_CGRP) * N_PER_W

    for c in range(COLS_W):
      pltpu.sync_copy(pet_hbm.at[c0 + c],
                      colbuf.at[pl.ds(c * TAB_PAD, TAB_PAD)])

    slots = ((idxb0, outb0, s_i0, s_o0), (idxb1, outb1, s_i1, s_o1))

    def idx_src(n):
      return idx_hbm.at[pl.ds(n * BATCH, BATCH)]

    def out_dst(n):
      return out_hbm.at[n, pl.ds(c0, COLS_W)]

    pltpu.async_copy(idx_src(n0), idxb0, s_i0)

    def pair(j, carry):
      for b in range(2):
        idxb, outb, s_i, s_o = slots[b]
        o_idxb, _, o_s_i, _ = slots[1 - b]
        n_i = 2 * j + b
        n = n0 + n_i

        pltpu.make_async_copy(idx_src(n), idxb, s_i).wait()

        if b == 0:
          pltpu.async_copy(idx_src(n + 1), o_idxb, o_s_i)
        else:
          @pl.when(j < N_PER_W // 2 - 1)
          def _():
            pltpu.async_copy(idx_src(n + 1), o_idxb, o_s_i)

        @pl.when(j >= 1)
        def _():
          pltpu.make_async_copy(outb, out_dst(n - 2), s_o).wait()

        @plsc.parallel_loop(0, BATCH, step=128, unroll=4)
        def _blk(base):
          for jj in range(8):
            iv = idxb[pl.ds(base + 16 * jj, 16)]
            ivc = [iv + jnp.int32(c * TAB_PAD) for c in range(COLS_W)]
            gs = [plsc.load_gather(colbuf, [ivc[c]]) for c in range(COLS_W)]
            for c in range(COLS_W):
              outb[c, 0, pl.ds(base + 16 * jj, 16)] = gs[c]
        pltpu.async_copy(outb, out_dst(n), s_o)
      return carry

    lax.fori_loop(0, N_PER_W // 2, pair, 0)

    pltpu.make_async_copy(outb0, out_dst(n0 + N_PER_W - 2), s_o0).wait()
    pltpu.make_async_copy(outb1, out_dst(n0 + N_PER_W - 1), s_o1).wait()

  return gather_kernel


_gather = _make_gather()


def kernel(doy, pe):
  idx_nb = doy.T.reshape(HIST * BATCH)
  pe_t = jnp.pad(pe.T, ((0, 0), (0, TAB_PAD - TAB_ROWS)))
  out = _gather(idx_nb, pe_t)
  return jnp.transpose(out, (3, 0, 1, 2))[:, :, :, :, None]
